# baseline (speedup 1.0000x reference)
import dataclasses
import functools

import jax
import jax.numpy as jnp
from jax import lax
from jax.experimental import pallas as pl
from jax.experimental.pallas import tpu as pltpu
from jax.experimental.pallas import tpu_sc as plsc

R = 128
C = 32768
L = 16
NSUB = 16
NW = 2 * NSUB
ROWS_PER_W = R // NW
GROUP = 8
GSIZE = GROUP * L
NGROUPS = C // GSIZE
IMAX = 2147483647

_mesh = plsc.VectorSubcoreMesh(core_axis_name="c", subcore_axis_name="s")

_cp = pltpu.CompilerParams()
if "needs_layout_passes" in getattr(pltpu.CompilerParams, "__dataclass_fields__", {}):
    _cp = dataclasses.replace(_cp, needs_layout_passes=False)


def _row_argmax(buf):
    iota = lax.iota(jnp.int32, L)

    def gbody(g, carry):
        best, bgrp = carry
        base = g * GSIZE
        gm = buf[pl.ds(base, L)]
        for k in range(1, GROUP):
            gm = jnp.maximum(gm, buf[pl.ds(base + k * L, L)])
        better = gm > best
        best = jnp.where(better, gm, best)
        bgrp = jnp.where(better, g, bgrp)
        return best, bgrp

    best, bgrp = lax.fori_loop(
        0, NGROUPS, gbody,
        (jnp.full((L,), -jnp.inf, jnp.float32), jnp.zeros((L,), jnp.int32)))

    m = jnp.max(best)
    gstar = jnp.min(jnp.where(best == m, bgrp, jnp.int32(IMAX)))
    base = gstar * GSIZE
    acc = jnp.full((L,), IMAX, jnp.int32)
    for k in range(GROUP):
        off = base + k * L
        v = buf[pl.ds(off, L)]
        acc = jnp.minimum(acc, jnp.where(v == m, iota + off, jnp.int32(IMAX)))
    return jnp.min(acc)


@functools.partial(
    pl.kernel,
    out_type=jax.ShapeDtypeStruct((R, C), jnp.float32),
    mesh=_mesh,
    scratch_types=[
        pltpu.VMEM((C,), jnp.float32),
        pltpu.VMEM((C,), jnp.float32),
        pltpu.VMEM((C,), jnp.float32),
        pltpu.SemaphoreType.DMA,
        pltpu.SemaphoreType.DMA,
        pltpu.SemaphoreType.DMA,
    ],
    compiler_params=_cp,
)
def _onehot_sc(x_hbm, o_hbm, buf0, buf1, ob, sem0, sem1, osem):
    wid = lax.axis_index("c") * NSUB + lax.axis_index("s")
    row0 = wid * ROWS_PER_W

    bufs = (buf0, buf1)
    sems = (sem0, sem1)
    in_cp = [None] * ROWS_PER_W
    in_cp[0] = pltpu.async_copy(x_hbm.at[row0], buf0, sem0)

    @pl.loop(0, C, step=GSIZE)
    def _(i):
        for k in range(GROUP):
            ob[pl.ds(i + k * L, L)] = jnp.zeros((L,), jnp.float32)

    iota = lax.iota(jnp.int32, L)
    prev_start = jnp.int32(0)
    out_cp = None
    for r in range(ROWS_PER_W):
        if r + 1 < ROWS_PER_W:
            in_cp[r + 1] = pltpu.async_copy(
                x_hbm.at[row0 + r + 1], bufs[(r + 1) % 2], sems[(r + 1) % 2])
        in_cp[r].wait()
        idx = _row_argmax(bufs[r % 2])
        start = (idx // L) * L
        pos = idx - start
        if out_cp is not None:
            out_cp.wait()
        ob[pl.ds(prev_start, L)] = jnp.zeros((L,), jnp.float32)
        ob[pl.ds(start, L)] = jnp.where(iota == pos, 1.0, 0.0).astype(jnp.float32)
        out_cp = pltpu.async_copy(ob, o_hbm.at[row0 + r], osem)
        prev_start = start
    out_cp.wait()


def kernel(x):
    return _onehot_sc(x)
